# baseline (speedup 1.0000x reference)
import functools

import jax
import jax.numpy as jnp
from jax import lax
from jax.experimental import pallas as pl
from jax.experimental.pallas import tpu as pltpu
from jax.experimental.pallas import tpu_sc as plsc

B = 64
M = 16384
CW = 64
R = 8
K = 8
IN = 512
C = K * R + 1
CP = 72
IFACE = CW * R + CW + 2

BLK = 2048
NB = M // BLK

_NEG = -3e38
_INTERPRET = False


def _iface_body(x_ref, w_ref, b_ref, q_ref, wv_ref, g_ref):
    xi = lax.dot_general(
        x_ref[...], w_ref[...], (((1,), (1,)), ((), ())),
        preferred_element_type=jnp.float32) + b_ref[...]
    q_ref[...] = xi[:, : R * CW]
    wv_ref[...] = jnp.tanh(xi[:, R * CW : R * CW + CW])
    ig = jax.nn.sigmoid(xi[:, R * CW + CW : R * CW + CW + 1])
    wg = jax.nn.sigmoid(xi[:, R * CW + CW + 1 : R * CW + CW + 2])
    g_ref[...] = jnp.concatenate(
        [ig, wg, jnp.zeros((B, 14), jnp.float32)], axis=1)


def _iface(x, W, b):
    return pl.pallas_call(
        _iface_body,
        out_shape=(
            jax.ShapeDtypeStruct((B, R * CW), jnp.float32),
            jax.ShapeDtypeStruct((B, CW), jnp.float32),
            jax.ShapeDtypeStruct((B, 16), jnp.float32),
        ),
        interpret=_INTERPRET,
    )(x, W, b.reshape(1, IFACE))


def _scan_body(q_ref, mem_ref, idx_ref, s_scr):
    m = pl.program_id(1)
    mem = mem_ref[0]
    s_scr[m] = lax.dot_general(
        q_ref[0], mem, (((1,), (1,)), ((), ())),
        preferred_element_type=jnp.float32)

    @pl.when(m == NB - 1)
    def _():
        cv = jnp.concatenate([s_scr[i] for i in range(NB)], axis=1)
        lane = lax.broadcasted_iota(jnp.int32, (R, M), 1)
        ps = []
        for _ in range(K):
            mx = jnp.max(cv, axis=1, keepdims=True)
            pos = jnp.min(jnp.where(cv >= mx, lane, jnp.int32(1 << 30)),
                          axis=1, keepdims=True)
            ps.append(pos)
            cv = jnp.where(lane == pos, _NEG, cv)
        idx_ref[0] = jnp.concatenate(ps, axis=1)


def _scan(q3, memory):
    return pl.pallas_call(
        _scan_body,
        grid=(B, NB),
        in_specs=[
            pl.BlockSpec((1, R, CW), lambda b, m: (b, 0, 0)),
            pl.BlockSpec((1, BLK, CW), lambda b, m: (b, m, 0)),
        ],
        out_specs=[
            pl.BlockSpec((1, R, K), lambda b, m: (b, 0, 0)),
        ],
        out_shape=(
            jax.ShapeDtypeStruct((B, R, K), jnp.int32),
        ),
        scratch_shapes=[
            pltpu.VMEM((NB, R, BLK), jnp.float32),
        ],
        interpret=_INTERPRET,
    )(q3, memory)


_NC = 2
_NS = 16
_NW = _NC * _NS
_BPW = B // _NW

_SC_PARAMS = pltpu.CompilerParams(
    needs_layout_passes=False, use_tc_tiling_on_sc=False)


def _gather_body(pos_hbm, mem_hbm, vis_hbm, pos_v, vis_v, sem):
    wid = lax.axis_index("s") * _NC + lax.axis_index("c")
    for j in range(_BPW):
        b = wid * _BPW + j
        pltpu.sync_copy(pos_hbm.at[b], pos_v)
        pltpu.async_copy(mem_hbm.at[pos_v], vis_v, sem).wait()
        pltpu.sync_copy(vis_v, vis_hbm.at[b])


def _sc_gather(pos, mem2d):
    mesh = plsc.VectorSubcoreMesh(core_axis_name="c", subcore_axis_name="s")
    f = pl.kernel(
        _gather_body,
        out_type=jax.ShapeDtypeStruct((B, CP, CW), jnp.float32),
        mesh=mesh,
        scratch_types=[
            pltpu.VMEM((CP,), jnp.int32),
            pltpu.VMEM((CP, CW), jnp.float32),
            pltpu.SemaphoreType.DMA,
        ],
        compiler_params=_SC_PARAMS,
    )
    return f(pos, mem2d)


_ROWS_PER_TILE = B * M // _NW


def _copy_scatter_body(pos_hbm, vn_hbm, mem_hbm, nm_hbm, pos_v, vn_v, sem):
    wid = lax.axis_index("s") * _NC + lax.axis_index("c")
    base = wid * _ROWS_PER_TILE
    pltpu.sync_copy(mem_hbm.at[pl.ds(base, _ROWS_PER_TILE)],
                    nm_hbm.at[pl.ds(base, _ROWS_PER_TILE)])
    for j in range(_BPW):
        b = wid * _BPW + j
        pltpu.sync_copy(pos_hbm.at[b], pos_v)
        pltpu.sync_copy(vn_hbm.at[b], vn_v)
        pltpu.async_copy(vn_v, nm_hbm.at[pos_v], sem).wait()


def _sc_copy_scatter(pos, vn, mem2d):
    mesh = plsc.VectorSubcoreMesh(core_axis_name="c", subcore_axis_name="s")
    f = pl.kernel(
        _copy_scatter_body,
        out_type=jax.ShapeDtypeStruct((B * M, CW), jnp.float32),
        mesh=mesh,
        scratch_types=[
            pltpu.VMEM((CP,), jnp.int32),
            pltpu.VMEM((CP, CW), jnp.float32),
            pltpu.SemaphoreType.DMA,
        ],
        compiler_params=_SC_PARAMS,
    )
    return f(pos, vn, mem2d)


def _attn_body(q_ref, vis_ref, wv_ref, g_ref, rv_ref, vn_ref):
    q = q_ref[0]
    vis = vis_ref[0]
    s = lax.dot_general(
        q, vis, (((1,), (1,)), ((), ())),
        preferred_element_type=jnp.float32)
    cols = lax.broadcasted_iota(jnp.int32, (R, CP), 1)
    s = jnp.where(cols < C, s, _NEG)
    mx = jnp.max(s, axis=1, keepdims=True)
    e = jnp.exp(s - mx)
    e = jnp.where(cols < C, e, 0.0)
    w = e / jnp.sum(e, axis=1, keepdims=True)
    rv_ref[0] = lax.dot_general(
        w, vis, (((1,), (0,)), ((), ())),
        preferred_element_type=jnp.float32)

    gv = g_ref[0, 0]
    ig = gv[0]
    wg = gv[1]
    ww = wg * (ig * jnp.mean(w, axis=0) + (1.0 - ig) / C)
    wvec = wv_ref[0, 0]
    vn = (vis * (1.0 - ww[:, None]) + ww[:, None] * wvec[None, :])
    row_lru = vn[C - 1 : C, :]
    rows = lax.broadcasted_iota(jnp.int32, (CP, CW), 0)
    vn_ref[0] = jnp.where(rows < C, vn, row_lru)


def _attn(q3, vis, wv, g):
    return pl.pallas_call(
        _attn_body,
        grid=(B,),
        in_specs=[
            pl.BlockSpec((1, R, CW), lambda b: (b, 0, 0)),
            pl.BlockSpec((1, CP, CW), lambda b: (b, 0, 0)),
            pl.BlockSpec((1, 1, CW), lambda b: (b, 0, 0)),
            pl.BlockSpec((1, 1, 16), lambda b: (b, 0, 0)),
        ],
        out_specs=[
            pl.BlockSpec((1, R, CW), lambda b: (b, 0, 0)),
            pl.BlockSpec((1, CP, CW), lambda b: (b, 0, 0)),
        ],
        out_shape=(
            jax.ShapeDtypeStruct((B, R, CW), jnp.float32),
            jax.ShapeDtypeStruct((B, CP, CW), jnp.float32),
        ),
        interpret=_INTERPRET,
    )(q3, vis, wv.reshape(B, 1, CW), g.reshape(B, 1, 16))


def kernel(x, memory, W, b):
    q, wv, g = _iface(x, W, b)
    q3 = q.reshape(B, R, CW)
    (idx,) = _scan(q3, memory)

    idxf = idx.reshape(B, R * K)
    pos = jnp.concatenate(
        [idxf, jnp.zeros((B, CP - R * K), jnp.int32)], axis=1)
    pos = pos + (jnp.arange(B, dtype=jnp.int32) * M)[:, None]

    mem2d = memory.reshape(B * M, CW)
    vis = _sc_gather(pos, mem2d)
    rv, vn = _attn(q3, vis, wv, g)

    nm2d = _sc_copy_scatter(pos, vn, mem2d)
    return rv.reshape(B, R * CW), nm2d.reshape(B, M, CW)


# speedup vs baseline: 4.5542x; 4.5542x over previous
import functools

import jax
import jax.numpy as jnp
from jax import lax
from jax.experimental import pallas as pl
from jax.experimental.pallas import tpu as pltpu
from jax.experimental.pallas import tpu_sc as plsc

B = 64
M = 16384
CW = 64
R = 8
K = 8
IN = 512
C = K * R + 1
CP = 72
IFACE = CW * R + CW + 2

BLK = 2048
NB = M // BLK

_NEG = -3e38
_INTERPRET = False


def _iface_body(x_ref, w_ref, b_ref, q_ref, wv_ref, g_ref):
    xi = lax.dot_general(
        x_ref[...], w_ref[...], (((1,), (1,)), ((), ())),
        preferred_element_type=jnp.float32) + b_ref[...]
    q_ref[...] = xi[:, : R * CW]
    wv_ref[...] = jnp.tanh(xi[:, R * CW : R * CW + CW])
    ig = jax.nn.sigmoid(xi[:, R * CW + CW : R * CW + CW + 1])
    wg = jax.nn.sigmoid(xi[:, R * CW + CW + 1 : R * CW + CW + 2])
    g_ref[...] = jnp.concatenate(
        [ig, wg, jnp.zeros((B, 14), jnp.float32)], axis=1)


def _iface(x, W, b):
    return pl.pallas_call(
        _iface_body,
        out_shape=(
            jax.ShapeDtypeStruct((B, R * CW), jnp.float32),
            jax.ShapeDtypeStruct((B, CW), jnp.float32),
            jax.ShapeDtypeStruct((B, 16), jnp.float32),
        ),
        interpret=_INTERPRET,
    )(x, W, b.reshape(1, IFACE))


def _scan_body(q_ref, mem_ref, idx_ref, s_scr):
    m = pl.program_id(1)
    mem = mem_ref[0]
    s_scr[m] = lax.dot_general(
        q_ref[0], mem, (((1,), (1,)), ((), ())),
        preferred_element_type=jnp.float32)

    @pl.when(m == NB - 1)
    def _():
        cv = jnp.concatenate([s_scr[i] for i in range(NB)], axis=1)
        lane = lax.broadcasted_iota(jnp.int32, (R, M), 1)
        ps = []
        for _ in range(K):
            mx = jnp.max(cv, axis=1, keepdims=True)
            pos = jnp.min(jnp.where(cv >= mx, lane, jnp.int32(1 << 30)),
                          axis=1, keepdims=True)
            ps.append(pos)
            cv = jnp.where(lane == pos, _NEG, cv)
        idx_ref[0] = jnp.concatenate(ps, axis=1)


def _scan(q3, memory):
    return pl.pallas_call(
        _scan_body,
        grid=(B, NB),
        in_specs=[
            pl.BlockSpec((1, R, CW), lambda b, m: (b, 0, 0)),
            pl.BlockSpec((1, BLK, CW), lambda b, m: (b, m, 0)),
        ],
        out_specs=[
            pl.BlockSpec((1, R, K), lambda b, m: (b, 0, 0)),
        ],
        out_shape=(
            jax.ShapeDtypeStruct((B, R, K), jnp.int32),
        ),
        scratch_shapes=[
            pltpu.VMEM((NB, R, BLK), jnp.float32),
        ],
        interpret=_INTERPRET,
    )(q3, memory)


_NC = 2
_NS = 16
_NW = _NC * _NS
_BPW = B // _NW

_SC_PARAMS = pltpu.CompilerParams(
    needs_layout_passes=False, use_tc_tiling_on_sc=False)


def _gather_body(pos_hbm, mem_hbm, vis_hbm, pos_v, vis_v, sem):
    wid = lax.axis_index("s") * _NC + lax.axis_index("c")
    for j in range(_BPW):
        b = wid * _BPW + j
        pltpu.sync_copy(pos_hbm.at[b], pos_v)
        pltpu.async_copy(mem_hbm.at[pos_v], vis_v, sem).wait()
        pltpu.sync_copy(vis_v, vis_hbm.at[b])


def _sc_gather(pos, mem2d):
    mesh = plsc.VectorSubcoreMesh(core_axis_name="c", subcore_axis_name="s")
    f = pl.kernel(
        _gather_body,
        out_type=jax.ShapeDtypeStruct((B, CP, CW), jnp.float32),
        mesh=mesh,
        scratch_types=[
            pltpu.VMEM((CP,), jnp.int32),
            pltpu.VMEM((CP, CW), jnp.float32),
            pltpu.SemaphoreType.DMA,
        ],
        compiler_params=_SC_PARAMS,
    )
    return f(pos, mem2d)


_ROWS_PER_TILE = B * M // _NW


_CCH = 512
_NCH = _ROWS_PER_TILE // _CCH


def _copy_scatter_body(pos_hbm, vn_hbm, mem_hbm, nm_hbm, pos_v, vn_v,
                       buf0, buf1, sem0, sem1, semw, sem):
    wid = lax.axis_index("s") * _NC + lax.axis_index("c")
    base = wid * _ROWS_PER_TILE
    bufs, sems = (buf0, buf1), (sem0, sem1)

    pltpu.async_copy(mem_hbm.at[pl.ds(base, _CCH)], buf0, sem0)

    def step(i, carry):
        for p in range(2):
            @pl.when(lax.rem(i, 2) == p)
            def _():
                pltpu.make_async_copy(
                    mem_hbm.at[pl.ds(base + i * _CCH, _CCH)],
                    bufs[p], sems[p]).wait()
                pltpu.async_copy(
                    bufs[p], nm_hbm.at[pl.ds(base + i * _CCH, _CCH)], semw)

                @pl.when(i + 1 < _NCH)
                def _():
                    pltpu.async_copy(
                        mem_hbm.at[pl.ds(base + (i + 1) * _CCH, _CCH)],
                        bufs[1 - p], sems[1 - p])

                pltpu.make_async_copy(
                    bufs[p], nm_hbm.at[pl.ds(base + i * _CCH, _CCH)],
                    semw).wait()
        return carry

    lax.fori_loop(0, _NCH, step, jnp.int32(0))

    for j in range(_BPW):
        b = wid * _BPW + j
        pltpu.sync_copy(pos_hbm.at[b], pos_v)
        pltpu.sync_copy(vn_hbm.at[b], vn_v)
        pltpu.async_copy(vn_v, nm_hbm.at[pos_v], sem).wait()


def _sc_copy_scatter(pos, vn, mem2d):
    mesh = plsc.VectorSubcoreMesh(core_axis_name="c", subcore_axis_name="s")
    f = pl.kernel(
        _copy_scatter_body,
        out_type=jax.ShapeDtypeStruct((B * M, CW), jnp.float32),
        mesh=mesh,
        scratch_types=[
            pltpu.VMEM((CP,), jnp.int32),
            pltpu.VMEM((CP, CW), jnp.float32),
            pltpu.VMEM((_CCH, CW), jnp.float32),
            pltpu.VMEM((_CCH, CW), jnp.float32),
            pltpu.SemaphoreType.DMA,
            pltpu.SemaphoreType.DMA,
            pltpu.SemaphoreType.DMA,
            pltpu.SemaphoreType.DMA,
        ],
        compiler_params=_SC_PARAMS,
    )
    return f(pos, vn, mem2d)


def _attn_body(q_ref, vis_ref, wv_ref, g_ref, rv_ref, vn_ref):
    q = q_ref[0]
    vis = vis_ref[0]
    s = lax.dot_general(
        q, vis, (((1,), (1,)), ((), ())),
        preferred_element_type=jnp.float32)
    cols = lax.broadcasted_iota(jnp.int32, (R, CP), 1)
    s = jnp.where(cols < C, s, _NEG)
    mx = jnp.max(s, axis=1, keepdims=True)
    e = jnp.exp(s - mx)
    e = jnp.where(cols < C, e, 0.0)
    w = e / jnp.sum(e, axis=1, keepdims=True)
    rv_ref[0] = lax.dot_general(
        w, vis, (((1,), (0,)), ((), ())),
        preferred_element_type=jnp.float32)

    gv = g_ref[0, 0]
    ig = gv[0]
    wg = gv[1]
    ww = wg * (ig * jnp.mean(w, axis=0) + (1.0 - ig) / C)
    wvec = wv_ref[0, 0]
    vn = (vis * (1.0 - ww[:, None]) + ww[:, None] * wvec[None, :])
    row_lru = vn[C - 1 : C, :]
    rows = lax.broadcasted_iota(jnp.int32, (CP, CW), 0)
    vn_ref[0] = jnp.where(rows < C, vn, row_lru)


def _attn(q3, vis, wv, g):
    return pl.pallas_call(
        _attn_body,
        grid=(B,),
        in_specs=[
            pl.BlockSpec((1, R, CW), lambda b: (b, 0, 0)),
            pl.BlockSpec((1, CP, CW), lambda b: (b, 0, 0)),
            pl.BlockSpec((1, 1, CW), lambda b: (b, 0, 0)),
            pl.BlockSpec((1, 1, 16), lambda b: (b, 0, 0)),
        ],
        out_specs=[
            pl.BlockSpec((1, R, CW), lambda b: (b, 0, 0)),
            pl.BlockSpec((1, CP, CW), lambda b: (b, 0, 0)),
        ],
        out_shape=(
            jax.ShapeDtypeStruct((B, R, CW), jnp.float32),
            jax.ShapeDtypeStruct((B, CP, CW), jnp.float32),
        ),
        interpret=_INTERPRET,
    )(q3, vis, wv.reshape(B, 1, CW), g.reshape(B, 1, 16))


def kernel(x, memory, W, b):
    q, wv, g = _iface(x, W, b)
    q3 = q.reshape(B, R, CW)
    (idx,) = _scan(q3, memory)

    idxf = idx.reshape(B, R * K)
    pos = jnp.concatenate(
        [idxf, jnp.zeros((B, CP - R * K), jnp.int32)], axis=1)
    pos = pos + (jnp.arange(B, dtype=jnp.int32) * M)[:, None]

    mem2d = memory.reshape(B * M, CW)
    vis = _sc_gather(pos, mem2d)
    rv, vn = _attn(q3, vis, wv, g)

    nm2d = _sc_copy_scatter(pos, vn, mem2d)
    return rv.reshape(B, R * CW), nm2d.reshape(B, M, CW)
